# Initial kernel scaffold; baseline (speedup 1.0000x reference)
import jax
import jax.numpy as jnp
from jax.experimental import pallas as pl


def kernel(x, params):
    raise NotImplementedError("write your pallas kernel here")


# speedup vs baseline: 8.9553x; 8.9553x over previous
import functools

import jax
import jax.numpy as jnp
from jax import lax
from jax.experimental import pallas as pl
from jax.experimental.pallas import tpu as pltpu
from jax.experimental.pallas import tpu_sc as plsc

KNN = 20
KPAD = 24


def _bn_scale_shift(p):
    s = p['g'] / jnp.sqrt(p['v'] + 1e-5)
    t = p['b'] - s * p['m']
    return s, t


def _prep_topk_body(x_ref, wc_ref, wd_ref, t_ref, idx_ref, a_ref, b_ref, *, n, k, kpad):
    b = pl.program_id(0)
    X = x_ref[0]
    a_ref[0] = jnp.dot(X, wc_ref[...], preferred_element_type=jnp.float32) + t_ref[...]
    b_ref[0] = jnp.dot(X, wd_ref[...], preferred_element_type=jnp.float32)
    G = lax.dot_general(X, X, (((1,), (1,)), ((), ())),
                        preferred_element_type=jnp.float32)
    xx = jnp.sum(X * X, axis=1, keepdims=True)
    work = xx + jnp.reshape(xx, (1, n)) - 2.0 * G
    iota = lax.broadcasted_iota(jnp.int32, (n, n), 1)
    base = b * n
    am = None
    for t in range(k):
        rowmin = jnp.min(work, axis=1, keepdims=True)
        cand = jnp.where(work == rowmin, iota, n)
        am = jnp.min(cand, axis=1)
        idx_ref[0, :, t] = am + base
        work = jnp.where(iota == am[:, None], jnp.inf, work)
    for t in range(k, kpad):
        idx_ref[0, :, t] = am + base


def _layer_tc(xT, wc_t, wd_t, tvec):
    B, N, Cp = xT.shape
    O = wc_t.shape[1]
    idx, Ap, Bp = pl.pallas_call(
        functools.partial(_prep_topk_body, n=N, k=KNN, kpad=KPAD),
        grid=(B,),
        in_specs=[
            pl.BlockSpec((1, N, Cp), lambda b: (b, 0, 0)),
            pl.BlockSpec((Cp, O), lambda b: (0, 0)),
            pl.BlockSpec((Cp, O), lambda b: (0, 0)),
            pl.BlockSpec((1, O), lambda b: (0, 0)),
        ],
        out_specs=[
            pl.BlockSpec((1, N, KPAD), lambda b: (b, 0, 0)),
            pl.BlockSpec((1, N, O), lambda b: (b, 0, 0)),
            pl.BlockSpec((1, N, O), lambda b: (b, 0, 0)),
        ],
        out_shape=[
            jax.ShapeDtypeStruct((B, N, KPAD), jnp.int32),
            jax.ShapeDtypeStruct((B, N, O), jnp.float32),
            jax.ShapeDtypeStruct((B, N, O), jnp.float32),
        ],
    )(xT, wc_t, wd_t, tvec[None, :])
    return idx, Ap, Bp


def _sc_gather_max(idx_flat, Ap, Bp):
    BN, O = Ap.shape
    WG = Bp.shape[1]
    K = KPAD
    CH = 4
    info = plsc.get_sparse_core_info()
    NC, NS = info.num_cores, info.num_subcores
    NW = NC * NS
    rows_per = BN // NW
    nch = rows_per // CH

    mesh = plsc.VectorSubcoreMesh(core_axis_name="c", subcore_axis_name="s")

    @functools.partial(
        pl.kernel, mesh=mesh,
        out_type=jax.ShapeDtypeStruct((BN, O), jnp.float32),
        scratch_types=[
            pltpu.VMEM((CH * K,), jnp.int32),
            pltpu.VMEM((CH * K,), jnp.int32),
            pltpu.VMEM((CH * K, WG), jnp.float32),
            pltpu.VMEM((CH * K, WG), jnp.float32),
            pltpu.VMEM((CH, O), jnp.float32),
            pltpu.VMEM((CH, O), jnp.float32),
            pltpu.SemaphoreType.DMA,
            pltpu.SemaphoreType.DMA,
        ],
    )
    def sck(idx_hbm, ap_hbm, bp_hbm, out_hbm, ib0, ib1, rb0, rb1, ab, ob, s0, s1):
        wid = lax.axis_index("s") * NC + lax.axis_index("c")
        base = wid * rows_per

        def issue(ch, ib, rb, sem):
            r0 = base + ch * CH
            pltpu.sync_copy(idx_hbm.at[pl.ds(r0 * K, CH * K)], ib)
            pltpu.async_copy(bp_hbm.at[ib], rb, sem)

        def wait(ib, rb, sem):
            pltpu.make_async_copy(bp_hbm.at[ib], rb, sem).wait()

        def compute(ch, rb):
            r0 = base + ch * CH
            pltpu.sync_copy(ap_hbm.at[pl.ds(r0, CH)], ab)
            for rr in range(CH):
                for co in range(O // 16):
                    sl = pl.ds(co * 16, 16)
                    a = ab[rr, sl]

                    def jb(j, acc):
                        v = rb[rr * K + j, sl]
                        y = a + v
                        return jnp.maximum(acc, jnp.maximum(y, 0.2 * y))

                    ob[rr, sl] = lax.fori_loop(
                        0, K, jb, jnp.full((16,), -jnp.inf, jnp.float32))
            pltpu.sync_copy(ob, out_hbm.at[pl.ds(r0, CH)])

        issue(0, ib0, rb0, s0)

        def body(i, carry):
            ch = 2 * i
            issue(ch + 1, ib1, rb1, s1)
            wait(ib0, rb0, s0)
            compute(ch, rb0)

            @pl.when(ch + 2 < nch)
            def _():
                issue(ch + 2, ib0, rb0, s0)

            wait(ib1, rb1, s1)
            compute(ch + 1, rb1)
            return carry

        lax.fori_loop(0, nch // 2, body, 0)

    return sck(idx_flat, Ap, Bp)


def _head_body(x1_ref, x2_ref, x3_ref, x4_ref, w1_ref, w2_ref, w3_ref, w4_ref,
               s5_ref, t5_ref, l1_ref, s6_ref, t6_ref,
               l2_ref, s7_ref, t7_ref, l3_ref, b3_ref, out_ref, *, n):
    dn = (((1,), (1,)), ((), ()))
    xe = (lax.dot_general(w1_ref[...], x1_ref[0], dn, preferred_element_type=jnp.float32)
          + lax.dot_general(w2_ref[...], x2_ref[0], dn, preferred_element_type=jnp.float32)
          + lax.dot_general(w3_ref[...], x3_ref[0], dn, preferred_element_type=jnp.float32)
          + lax.dot_general(w4_ref[...], x4_ref[0], dn, preferred_element_type=jnp.float32))
    xe = xe * s5_ref[...].T + t5_ref[...].T
    xe = jnp.maximum(xe, 0.2 * xe)
    xm = jnp.max(xe, axis=1)
    xa = jnp.sum(xe, axis=1) * (1.0 / n)
    xf = jnp.concatenate([xm, xa], axis=0)[None, :]
    h = jnp.dot(xf, l1_ref[...].T, preferred_element_type=jnp.float32) * s6_ref[...] + t6_ref[...]
    h = jnp.maximum(h, 0.2 * h)
    h = jnp.dot(h, l2_ref[...].T, preferred_element_type=jnp.float32) * s7_ref[...] + t7_ref[...]
    h = jnp.maximum(h, 0.2 * h)
    out_ref[0] = jnp.dot(h, l3_ref[...].T, preferred_element_type=jnp.float32) + b3_ref[...]


def _head(xs, params):
    B, N, _ = xs[0].shape
    s5, t5 = _bn_scale_shift(params['bn5'])
    s6, t6 = _bn_scale_shift(params['bn6'])
    s7, t7 = _bn_scale_shift(params['bn7'])
    W5 = params['W5']
    w5s = (W5[:, :64], W5[:, 64:128], W5[:, 128:256], W5[:, 256:512])
    in_specs = [pl.BlockSpec((1, N, xs[i].shape[2]), lambda b: (b, 0, 0)) for i in range(4)]
    in_specs += [pl.BlockSpec(w.shape, lambda b: tuple(0 for _ in w.shape)) for w in w5s]
    scalars = [s5[:, None], t5[:, None], params['L1'], s6[None, :], t6[None, :],
               params['L2'], s7[None, :], t7[None, :], params['L3'], params['L3b'][None, :]]
    in_specs += [pl.BlockSpec(a.shape, lambda b: tuple(0 for _ in a.shape)) for a in scalars]
    out = pl.pallas_call(
        functools.partial(_head_body, n=N),
        grid=(B,),
        in_specs=in_specs,
        out_specs=pl.BlockSpec((1, 1, 40), lambda b: (b, 0, 0)),
        out_shape=jax.ShapeDtypeStruct((B, 1, 40), jnp.float32),
    )(*xs, *w5s, *scalars)
    return out[:, 0, :]


def _layer(xT, W, bnp):
    B, N, Cp = xT.shape
    C2 = W.shape[1]
    C = C2 // 2
    s, t = _bn_scale_shift(bnp)
    Wc = (W[:, :C] - W[:, C:]) * s[:, None]
    Wd = W[:, C:] * s[:, None]
    if C < Cp:
        Wc = jnp.pad(Wc, ((0, 0), (0, Cp - C)))
        Wd = jnp.pad(Wd, ((0, 0), (0, Cp - C)))
    idx, Ap, Bp = _layer_tc(xT, Wc.T, Wd.T, t)
    O = Wc.shape[0]
    Bp2 = Bp.reshape(B * N, O)
    if O < 128:
        Bp2 = jnp.pad(Bp2, ((0, 0), (0, 128 - O)))
    out = _sc_gather_max(idx.reshape(-1), Ap.reshape(B * N, O), Bp2)
    return out.reshape(B, N, O)


def kernel(x, params):
    B, C0, N = x.shape
    xT = jnp.pad(jnp.swapaxes(x, 1, 2), ((0, 0), (0, 0), (0, 8 - C0)))
    x1 = _layer(xT, params['W1'], params['bn1'])
    x2 = _layer(x1, params['W2'], params['bn2'])
    x3 = _layer(x2, params['W3'], params['bn3'])
    x4 = _layer(x3, params['W4'], params['bn4'])
    return _head((x1, x2, x3, x4), params)


# speedup vs baseline: 10.7356x; 1.1988x over previous
import functools

import jax
import jax.numpy as jnp
from jax import lax
from jax.experimental import pallas as pl
from jax.experimental.pallas import tpu as pltpu
from jax.experimental.pallas import tpu_sc as plsc

KNN = 20
KPAD = 24


def _bn_scale_shift(p):
    s = p['g'] / jnp.sqrt(p['v'] + 1e-5)
    t = p['b'] - s * p['m']
    return s, t


def _prep_topk_body(x_ref, wc_ref, wd_ref, t_ref, idx_ref, a_ref, b_ref, *, n, k, kpad):
    b = pl.program_id(0)
    X = x_ref[0]
    a_ref[0] = jnp.dot(X, wc_ref[...], preferred_element_type=jnp.float32) + t_ref[...]
    b_ref[0] = jnp.dot(X, wd_ref[...], preferred_element_type=jnp.float32)
    G = lax.dot_general(X, X, (((1,), (1,)), ((), ())),
                        preferred_element_type=jnp.float32)
    xx = jnp.sum(X * X, axis=1, keepdims=True)
    work = xx + jnp.reshape(xx, (1, n)) - 2.0 * G
    iota = lax.broadcasted_iota(jnp.int32, (n, n), 1)
    base = b * n
    am = None
    for t in range(k):
        rowmin = jnp.min(work, axis=1, keepdims=True)
        cand = jnp.where(work == rowmin, iota, n)
        am = jnp.min(cand, axis=1)
        idx_ref[0, :, t] = am + base
        work = jnp.where(iota == am[:, None], jnp.inf, work)
    for t in range(k, kpad):
        idx_ref[0, :, t] = am + base


def _layer_tc(xT, wc_t, wd_t, tvec):
    B, N, Cp = xT.shape
    O = wc_t.shape[1]
    idx, Ap, Bp = pl.pallas_call(
        functools.partial(_prep_topk_body, n=N, k=KNN, kpad=KPAD),
        grid=(B,),
        in_specs=[
            pl.BlockSpec((1, N, Cp), lambda b: (b, 0, 0)),
            pl.BlockSpec((Cp, O), lambda b: (0, 0)),
            pl.BlockSpec((Cp, O), lambda b: (0, 0)),
            pl.BlockSpec((1, O), lambda b: (0, 0)),
        ],
        out_specs=[
            pl.BlockSpec((1, N, KPAD), lambda b: (b, 0, 0)),
            pl.BlockSpec((1, N, O), lambda b: (b, 0, 0)),
            pl.BlockSpec((1, N, O), lambda b: (b, 0, 0)),
        ],
        out_shape=[
            jax.ShapeDtypeStruct((B, N, KPAD), jnp.int32),
            jax.ShapeDtypeStruct((B, N, O), jnp.float32),
            jax.ShapeDtypeStruct((B, N, O), jnp.float32),
        ],
    )(xT, wc_t, wd_t, tvec[None, :])
    return idx, Ap, Bp


def _sc_gather_max(idx_flat, Ap, Bp):
    BN, O = Ap.shape
    WG = Bp.shape[1]
    K = KPAD
    CH = 4
    info = plsc.get_sparse_core_info()
    NC, NS = info.num_cores, info.num_subcores
    NW = NC * NS
    rows_per = BN // NW
    nch = rows_per // CH

    mesh = plsc.VectorSubcoreMesh(core_axis_name="c", subcore_axis_name="s")

    @functools.partial(
        pl.kernel, mesh=mesh,
        out_type=jax.ShapeDtypeStruct((BN, O), jnp.float32),
        scratch_types=[
            pltpu.VMEM((CH * K,), jnp.int32),
            pltpu.VMEM((CH * K,), jnp.int32),
            pltpu.VMEM((CH * K, WG), jnp.float32),
            pltpu.VMEM((CH * K, WG), jnp.float32),
            pltpu.VMEM((CH, O), jnp.float32),
            pltpu.VMEM((CH, O), jnp.float32),
            pltpu.VMEM((CH, O), jnp.float32),
            pltpu.SemaphoreType.DMA,
            pltpu.SemaphoreType.DMA,
        ],
    )
    def sck(idx_hbm, ap_hbm, bp_hbm, out_hbm, ib0, ib1, rb0, rb1, av0, av1, ob, s0, s1):
        wid = lax.axis_index("s") * NC + lax.axis_index("c")
        base = wid * rows_per

        def issue(ch, ib, rb, av, sem):
            r0 = base + ch * CH
            pltpu.sync_copy(idx_hbm.at[pl.ds(r0 * K, CH * K)], ib)
            pltpu.async_copy(bp_hbm.at[ib], rb, sem)
            pltpu.async_copy(ap_hbm.at[pl.ds(r0, CH)], av, sem)

        def wait(ib, rb, av, sem):
            pltpu.make_async_copy(bp_hbm.at[ib], rb, sem).wait()
            pltpu.make_async_copy(ap_hbm.at[pl.ds(0, CH)], av, sem).wait()

        def compute(ch, rb, av):
            r0 = base + ch * CH
            for rr in range(CH):
                for co in range(O // 16):
                    sl = pl.ds(co * 16, 16)

                    def jb(j, acc):
                        return jnp.maximum(acc, rb[rr * K + j, sl])

                    acc = lax.fori_loop(
                        0, K, jb, jnp.full((16,), -jnp.inf, jnp.float32))
                    y = av[rr, sl] + acc
                    ob[rr, sl] = jnp.maximum(y, 0.2 * y)
            pltpu.sync_copy(ob, out_hbm.at[pl.ds(r0, CH)])

        issue(0, ib0, rb0, av0, s0)

        def body(i, carry):
            ch = 2 * i
            issue(ch + 1, ib1, rb1, av1, s1)
            wait(ib0, rb0, av0, s0)
            compute(ch, rb0, av0)

            @pl.when(ch + 2 < nch)
            def _():
                issue(ch + 2, ib0, rb0, av0, s0)

            wait(ib1, rb1, av1, s1)
            compute(ch + 1, rb1, av1)
            return carry

        lax.fori_loop(0, nch // 2, body, 0)

    return sck(idx_flat, Ap, Bp)


def _head_body(x1_ref, x2_ref, x3_ref, x4_ref, w1_ref, w2_ref, w3_ref, w4_ref,
               s5_ref, t5_ref, l1_ref, s6_ref, t6_ref,
               l2_ref, s7_ref, t7_ref, l3_ref, b3_ref, out_ref, *, n):
    dn = (((1,), (1,)), ((), ()))
    xe = (lax.dot_general(w1_ref[...], x1_ref[0], dn, preferred_element_type=jnp.float32)
          + lax.dot_general(w2_ref[...], x2_ref[0], dn, preferred_element_type=jnp.float32)
          + lax.dot_general(w3_ref[...], x3_ref[0], dn, preferred_element_type=jnp.float32)
          + lax.dot_general(w4_ref[...], x4_ref[0], dn, preferred_element_type=jnp.float32))
    xe = xe * s5_ref[...].T + t5_ref[...].T
    xe = jnp.maximum(xe, 0.2 * xe)
    xm = jnp.max(xe, axis=1)
    xa = jnp.sum(xe, axis=1) * (1.0 / n)
    xf = jnp.concatenate([xm, xa], axis=0)[None, :]
    h = jnp.dot(xf, l1_ref[...].T, preferred_element_type=jnp.float32) * s6_ref[...] + t6_ref[...]
    h = jnp.maximum(h, 0.2 * h)
    h = jnp.dot(h, l2_ref[...].T, preferred_element_type=jnp.float32) * s7_ref[...] + t7_ref[...]
    h = jnp.maximum(h, 0.2 * h)
    out_ref[0] = jnp.dot(h, l3_ref[...].T, preferred_element_type=jnp.float32) + b3_ref[...]


def _head(xs, params):
    B, N, _ = xs[0].shape
    s5, t5 = _bn_scale_shift(params['bn5'])
    s6, t6 = _bn_scale_shift(params['bn6'])
    s7, t7 = _bn_scale_shift(params['bn7'])
    W5 = params['W5']
    w5s = (W5[:, :64], W5[:, 64:128], W5[:, 128:256], W5[:, 256:512])
    in_specs = [pl.BlockSpec((1, N, xs[i].shape[2]), lambda b: (b, 0, 0)) for i in range(4)]
    in_specs += [pl.BlockSpec(w.shape, lambda b: tuple(0 for _ in w.shape)) for w in w5s]
    scalars = [s5[:, None], t5[:, None], params['L1'], s6[None, :], t6[None, :],
               params['L2'], s7[None, :], t7[None, :], params['L3'], params['L3b'][None, :]]
    in_specs += [pl.BlockSpec(a.shape, lambda b: tuple(0 for _ in a.shape)) for a in scalars]
    out = pl.pallas_call(
        functools.partial(_head_body, n=N),
        grid=(B,),
        in_specs=in_specs,
        out_specs=pl.BlockSpec((1, 1, 40), lambda b: (b, 0, 0)),
        out_shape=jax.ShapeDtypeStruct((B, 1, 40), jnp.float32),
    )(*xs, *w5s, *scalars)
    return out[:, 0, :]


def _layer(xT, W, bnp):
    B, N, Cp = xT.shape
    C2 = W.shape[1]
    C = C2 // 2
    s, t = _bn_scale_shift(bnp)
    Wc = (W[:, :C] - W[:, C:]) * s[:, None]
    Wd = W[:, C:] * s[:, None]
    if C < Cp:
        Wc = jnp.pad(Wc, ((0, 0), (0, Cp - C)))
        Wd = jnp.pad(Wd, ((0, 0), (0, Cp - C)))
    idx, Ap, Bp = _layer_tc(xT, Wc.T, Wd.T, t)
    O = Wc.shape[0]
    Bp2 = Bp.reshape(B * N, O)
    if O < 128:
        Bp2 = jnp.pad(Bp2, ((0, 0), (0, 128 - O)))
    out = _sc_gather_max(idx.reshape(-1), Ap.reshape(B * N, O), Bp2)
    return out.reshape(B, N, O)


def kernel(x, params):
    B, C0, N = x.shape
    xT = jnp.pad(jnp.swapaxes(x, 1, 2), ((0, 0), (0, 0), (0, 8 - C0)))
    x1 = _layer(xT, params['W1'], params['bn1'])
    x2 = _layer(x1, params['W2'], params['bn2'])
    x3 = _layer(x2, params['W3'], params['bn3'])
    x4 = _layer(x3, params['W4'], params['bn4'])
    return _head((x1, x2, x3, x4), params)
